# baseline (speedup 1.0000x reference)
import jax
import jax.numpy as jnp
import numpy as np
from jax.experimental import pallas as pl

N = 10000
B = 16


def _copy_body(x_ref, o_ref):
    o_ref[...] = x_ref[...]


def kernel(x, edges, query, batch, W_gat, att_src, att_dst, b_gat, W_att, W_lin, b_lin):
    ar = jnp.arange(N, dtype=edges.dtype)
    ei = jnp.concatenate([edges, jnp.stack([ar, ar])], axis=1)
    src, dst = ei[0], ei[1]
    h = x
    for _ in range(3):
        hp = h @ W_gat.T
        a_s = hp @ att_src
        a_d = hp @ att_dst
        e = jax.nn.leaky_relu(a_s[src] + a_d[dst], negative_slope=0.2)
        m = jax.ops.segment_max(e, dst, num_segments=N)
        m = jnp.where(jnp.isfinite(m), m, 0.0)
        ex = jnp.exp(e - m[dst])
        s = jax.ops.segment_sum(ex, dst, num_segments=N)
        alpha = ex / (s[dst] + 1e-16)
        out = jax.ops.segment_sum(alpha[:, None] * hp[src], dst, num_segments=N)
        h = jax.nn.relu(out + b_gat)
    proj = h @ W_att.T
    att = (proj @ query.T) / np.sqrt(query.shape[1])
    m = jax.ops.segment_max(att, batch, num_segments=B)
    m = jnp.where(jnp.isfinite(m), m, 0.0)
    ex = jnp.exp(att - m[batch])
    ssum = jax.ops.segment_sum(ex, batch, num_segments=B)
    w_full = ex / (ssum[batch] + 1e-16)
    sel = w_full[jnp.arange(N), batch]
    pooled = jax.ops.segment_sum(sel[:, None] * h, batch, num_segments=B)
    pooled = jax.nn.relu(pooled)
    out = pooled @ W_lin.T + b_lin
    return pl.pallas_call(
        _copy_body, out_shape=jax.ShapeDtypeStruct(out.shape, out.dtype)
    )(out)


# speedup vs baseline: 9.9033x; 9.9033x over previous
import functools

import jax
import jax.numpy as jnp
import numpy as np
from jax import lax
from jax.experimental import pallas as pl
from jax.experimental.pallas import tpu as pltpu
from jax.experimental.pallas import tpu_sc as plsc

N = 10000
E = 160000
F_FEAT = 302
Q_DIM = 600
B = 16
CLASSES = 100

FP = 304
HALF = 5000
HALFP = 5120
NP = 2 * HALFP
EP = E + N
CHUNK = 10640
EPP = 16 * CHUNK
NB_RED = HALFP // (16 * 16)
SEG = HALFP // 16
SUB = 2128
RT = 512
GRID_N = NP // RT


def _shift_up(v, sh):
    j = jnp.maximum(lax.iota(jnp.int32, 16) - sh, 0)
    return v.at[j].get(mode="promise_in_bounds")


def _sc_edge_body(ep_hbm, as_hbm, ad_hbm, h_hbm,
                  out_hbm, s_hbm, ce_hbm,
                  es_pack, cnts,
                  as_loc, ad_loc, m_loc, acc_priv, tmpseg, red320,
                  rowbuf, sidx, didx,
                  out_sh, red_small,
                  sem):
    c = lax.axis_index("c")
    t = lax.axis_index("s")
    cbase = c * HALFP
    seg0 = t * SEG
    cebase = (c * 16 + t) * CHUNK
    iota16 = lax.iota(jnp.int32, 16)
    f32 = jnp.float32

    pltpu.sync_copy(as_hbm, as_loc)
    pltpu.sync_copy(ad_hbm.at[pl.ds(cbase, HALFP)], ad_loc)

    def _zneg(i, _):
        acc_priv[pl.ds(i * 16, 16)] = jnp.full((16,), -1e30, f32)
        return 0
    lax.fori_loop(0, HALFP // 16, _zneg, 0)

    def _zrow(i, _):
        rowbuf[i // (FP // 16), pl.ds((i % (FP // 16)) * 16, 16)] = jnp.zeros((16,), f32)
        return 0
    lax.fori_loop(0, 16 * (FP // 16), _zrow, 0)

    def _zout(k, _):
        pltpu.sync_copy(rowbuf, out_sh.at[pl.ds(seg0 + k * 16, 16)])
        return 0
    lax.fori_loop(0, SEG // 16, _zout, 0)

    def _psub(pno, _):
        pltpu.sync_copy(ep_hbm.at[pl.ds(t * CHUNK + pno * SUB, SUB)], es_pack)

        def _part(blk, fc):
            pk = es_pack[pl.ds(blk * 16, 16)]
            dl = (pk >> 14) - cbase
            msk = (dl >= 0) & (dl < HALFP)
            mi = msk.astype(jnp.int32)
            pos = fc + plsc.cumsum(mi) - 1
            plsc.store_scatter(es_pack, [pos], pk, mask=msk)
            return fc + jnp.sum(mi)
        fc = lax.fori_loop(0, SUB // 16, _part, jnp.int32(0))
        plsc.store_scatter(cnts, [jnp.full((16,), pno, jnp.int32)],
                           jnp.full((16,), fc, jnp.int32), mask=iota16 == 0)
        pltpu.sync_copy(es_pack, ce_hbm.at[pl.ds(cebase + pno * SUB, SUB)])
        return 0
    lax.fori_loop(0, CHUNK // SUB, _psub, 0)
    cvec = cnts[...]

    def _p2sub(pno, _):
        fcp = jnp.sum(jnp.where(iota16 == pno, cvec, 0))
        pltpu.sync_copy(ce_hbm.at[pl.ds(cebase + pno * SUB, SUB)], es_pack)

        def _p2(blk, _):
            off = blk * 16
            lm = (off + iota16) < fcp
            pk = es_pack[pl.ds(off, 16)]
            pk = jnp.where(lm, pk, cbase * 16384)
            sl = pk & 16383
            dl = (pk >> 14) - cbase
            av = plsc.load_gather(as_loc, [sl])
            ks, vs, om = plsc.sort_key_val(dl, av, mask=lm)
            for sh in (1, 2, 4, 8):
                kj = _shift_up(ks, sh)
                vj = _shift_up(vs, sh)
                same = (kj == ks) & (iota16 >= sh)
                vs = jnp.where(same, jnp.maximum(vs, vj), vs)
            _, lastm = plsc.scan_count(ks, mask=om)
            kw = jnp.where(lastm, ks, 0)
            cur = plsc.load_gather(acc_priv, [kw])
            plsc.store_scatter(acc_priv, [kw], jnp.maximum(cur, vs), mask=lastm)
            return 0
        lax.fori_loop(0, (fcp + 15) // 16, _p2, 0)
        return 0
    lax.fori_loop(0, CHUNK // SUB, _p2sub, 0)

    def _zr(i, _):
        red320[pl.ds(i * 16, 16)] = jnp.full((16,), -1e30, f32)
        return 0
    lax.fori_loop(0, NB_RED, _zr, 0)

    def _rring(r, _):
        sseg = lax.rem(t + r, 16)
        pltpu.sync_copy(acc_priv.at[pl.ds(sseg * SEG, SEG)],
                        red_small.at[pl.ds(sseg * SEG, SEG)])
        plsc.subcore_barrier()
        pltpu.sync_copy(red_small.at[pl.ds(t * SEG, SEG)], tmpseg)

        def _acc(j, _):
            col = pl.ds(j * 16, 16)
            red320[col] = jnp.maximum(red320[col], tmpseg[col])
            return 0
        lax.fori_loop(0, NB_RED, _acc, 0)
        plsc.subcore_barrier()
        return 0
    lax.fori_loop(0, 16, _rring, 0)

    def _mfin(j, _):
        col = pl.ds(j * 16, 16)
        x = ad_loc[pl.ds(seg0 + j * 16, 16)] + red320[col]
        red320[col] = jnp.where(x >= 0, x, 0.2 * x)
        return 0
    lax.fori_loop(0, NB_RED, _mfin, 0)
    plsc.subcore_barrier()
    pltpu.sync_copy(red320, red_small.at[pl.ds(seg0, SEG)])
    plsc.subcore_barrier()
    pltpu.sync_copy(red_small, m_loc)

    def _zzero(i, _):
        acc_priv[pl.ds(i * 16, 16)] = jnp.zeros((16,), f32)
        return 0
    lax.fori_loop(0, HALFP // 16, _zzero, 0)

    def _p4sub(pno, _):
        fcp = jnp.sum(jnp.where(iota16 == pno, cvec, 0))
        pltpu.sync_copy(ce_hbm.at[pl.ds(cebase + pno * SUB, SUB)], es_pack)

        def _p4(b, _):
            off = b * 16
            lm = (off + iota16) < fcp
            pk = es_pack[pl.ds(off, 16)]
            pk = jnp.where(lm, pk, cbase * 16384)
            sl = pk & 16383
            dl = (pk >> 14) - cbase
            a1 = plsc.load_gather(as_loc, [sl])
            a2 = plsc.load_gather(ad_loc, [dl])
            mm = plsc.load_gather(m_loc, [dl])
            xx = a1 + a2
            e = jnp.where(xx >= 0, xx, 0.2 * xx)
            ex = jnp.where(lm, jnp.exp(e - mm), 0.0)
            ks, vs, om = plsc.sort_key_val(dl, ex, mask=lm)
            for sh in (1, 2, 4, 8):
                kj = _shift_up(ks, sh)
                vj = _shift_up(vs, sh)
                same = (kj == ks) & (iota16 >= sh)
                vs = jnp.where(same, vs + vj, vs)
            _, lastm = plsc.scan_count(ks, mask=om)
            kw = jnp.where(lastm, ks, 0)
            cur = plsc.load_gather(acc_priv, [kw])
            plsc.store_scatter(acc_priv, [kw], cur + vs, mask=lastm)
            sidx[...] = sl
            didx[...] = dl
            pltpu.async_copy(h_hbm.at[sidx], rowbuf, sem).wait()
            for r in range(16):
                er = ex[r]
                for jj in range(FP // 16):
                    slc = pl.ds(jj * 16, 16)
                    rowbuf[r, slc] = rowbuf[r, slc] * er
            pltpu.sync_copy(rowbuf, out_sh.at[didx], add=True)
            return 0
        lax.fori_loop(0, (fcp + 15) // 16, _p4, 0)
        return 0
    lax.fori_loop(0, CHUNK // SUB, _p4sub, 0)

    def _zs(i, _):
        red320[pl.ds(i * 16, 16)] = jnp.zeros((16,), f32)
        return 0
    lax.fori_loop(0, NB_RED, _zs, 0)

    def _sring(r, _):
        sseg = lax.rem(t + r, 16)
        pltpu.sync_copy(acc_priv.at[pl.ds(sseg * SEG, SEG)],
                        red_small.at[pl.ds(sseg * SEG, SEG)])
        plsc.subcore_barrier()
        pltpu.sync_copy(red_small.at[pl.ds(t * SEG, SEG)], tmpseg)

        def _acc(j, _):
            col = pl.ds(j * 16, 16)
            red320[col] = red320[col] + tmpseg[col]
            return 0
        lax.fori_loop(0, NB_RED, _acc, 0)
        plsc.subcore_barrier()
        return 0
    lax.fori_loop(0, 16, _sring, 0)
    pltpu.sync_copy(red320, s_hbm.at[pl.ds(cbase + seg0, SEG)])

    plsc.subcore_barrier()
    pltpu.sync_copy(out_sh.at[pl.ds(seg0, SEG)],
                    out_hbm.at[pl.ds(cbase + seg0, SEG)])


_sc_edge = functools.partial(
    pl.kernel,
    out_type=[
        jax.ShapeDtypeStruct((NP, FP), jnp.float32),
        jax.ShapeDtypeStruct((NP,), jnp.float32),
        jax.ShapeDtypeStruct((32 * CHUNK,), jnp.int32),
    ],
    mesh=plsc.VectorSubcoreMesh(
        core_axis_name="c", subcore_axis_name="s", num_cores=2, num_subcores=16
    ),
    compiler_params=pltpu.CompilerParams(
        needs_layout_passes=False, use_tc_tiling_on_sc=False),
    scratch_types=[
        pltpu.VMEM((SUB,), jnp.int32),
        pltpu.VMEM((16,), jnp.int32),
        pltpu.VMEM((NP,), jnp.float32),
        pltpu.VMEM((HALFP,), jnp.float32),
        pltpu.VMEM((HALFP,), jnp.float32),
        pltpu.VMEM((HALFP,), jnp.float32),
        pltpu.VMEM((SEG,), jnp.float32),
        pltpu.VMEM((SEG,), jnp.float32),
        pltpu.VMEM((16, FP), jnp.float32),
        pltpu.VMEM((16,), jnp.int32),
        pltpu.VMEM((16,), jnp.int32),
        pltpu.VMEM_SHARED((HALFP, FP), jnp.float32),
        pltpu.VMEM_SHARED((16 * SEG,), jnp.float32),
        pltpu.SemaphoreType.DMA,
    ],
)(_sc_edge_body)


def _mm_first_body(x_ref, w_ref, av_ref, ad_ref, h_ref, as_ref, ad_out_ref):
    y = jnp.dot(x_ref[...], w_ref[...], preferred_element_type=jnp.float32)
    h_ref[...] = y
    as_ref[...] = jnp.sum(y * av_ref[...], axis=1)
    ad_out_ref[...] = jnp.sum(y * ad_ref[...], axis=1)


def _mm_next_body(raw_ref, s_ref, b_ref, w_ref, av_ref, ad_ref,
                  h_ref, as_ref, ad_out_ref):
    hin = jnp.maximum(
        raw_ref[...] / (s_ref[...][:, None] + 1e-16) + b_ref[...], 0.0)
    y = jnp.dot(hin, w_ref[...], preferred_element_type=jnp.float32)
    h_ref[...] = y
    as_ref[...] = jnp.sum(y * av_ref[...], axis=1)
    ad_out_ref[...] = jnp.sum(y * ad_ref[...], axis=1)


def _pool1_body(raw_ref, s_ref, b_ref, wt_ref, qt_ref, batch_ref,
                h3_ref, t_ref, m_ref, qw_scratch):
    i = pl.program_id(0)

    @pl.when(i == 0)
    def _():
        qw_scratch[...] = jnp.dot(
            wt_ref[...], qt_ref[...], preferred_element_type=jnp.float32
        ) * (1.0 / np.sqrt(Q_DIM))

    hin = jnp.maximum(
        raw_ref[...] / (s_ref[...][:, None] + 1e-16) + b_ref[...], 0.0)
    h3_ref[...] = hin
    attq = jnp.dot(hin, qw_scratch[...], preferred_element_type=jnp.float32)
    gids = lax.broadcasted_iota(jnp.int32, (RT, B), 1)
    oh = batch_ref[...][:, None] == gids
    t_ref[...] = jnp.sum(jnp.where(oh, attq, 0.0), axis=1)
    mt = jnp.max(jnp.where(oh, attq, -3.0e38), axis=0, keepdims=True)

    @pl.when(i == 0)
    def _():
        m_ref[...] = jnp.full((1, B), -3.0e38, jnp.float32)

    m_ref[...] = jnp.maximum(m_ref[...], mt)


def _pool2_body(h3_ref, t_ref, batch_ref, m_ref, wl_ref, bl_ref,
                logits_ref, pooled_sc, ssum_sc):
    i = pl.program_id(0)

    @pl.when(i == 0)
    def _():
        pooled_sc[...] = jnp.zeros((B, FP), jnp.float32)
        ssum_sc[...] = jnp.zeros((B, 1), jnp.float32)

    gids = lax.broadcasted_iota(jnp.int32, (B, RT), 0)
    ohf = (gids == batch_ref[...][None, :]).astype(jnp.float32)
    m_n = jnp.sum(ohf * m_ref[...].reshape(B, 1), axis=0)
    w_u = jnp.exp(t_ref[...] - m_n)
    ohw = ohf * w_u[None, :]
    ssum_sc[...] += jnp.sum(ohw, axis=1, keepdims=True)
    pooled_sc[...] += jnp.dot(ohw, h3_ref[...], preferred_element_type=jnp.float32)

    @pl.when(i == pl.num_programs(0) - 1)
    def _():
        pooled = jnp.maximum(pooled_sc[...] / (ssum_sc[...] + 1e-16), 0.0)
        logits_ref[...] = jnp.dot(
            pooled, wl_ref[...], preferred_element_type=jnp.float32
        ) + bl_ref[...]


def _row_spec():
    return pl.BlockSpec((RT, FP), lambda i: (i, 0))


def _vec_spec():
    return pl.BlockSpec((RT,), lambda i: (i,))


def _full_spec(shape):
    nd = len(shape)
    return pl.BlockSpec(shape, lambda i: (0,) * nd)


def _mm_first(xp, wp, av2, ad2):
    return pl.pallas_call(
        _mm_first_body,
        grid=(GRID_N,),
        in_specs=[_row_spec(), _full_spec((FP, FP)), _full_spec((1, FP)),
                  _full_spec((1, FP))],
        out_specs=[_row_spec(), _vec_spec(), _vec_spec()],
        out_shape=[
            jax.ShapeDtypeStruct((NP, FP), jnp.float32),
            jax.ShapeDtypeStruct((NP,), jnp.float32),
            jax.ShapeDtypeStruct((NP,), jnp.float32),
        ],
    )(xp, wp, av2, ad2)


def _mm_next(raw, s, b2, wp, av2, ad2):
    return pl.pallas_call(
        _mm_next_body,
        grid=(GRID_N,),
        in_specs=[_row_spec(), _vec_spec(), _full_spec((1, FP)),
                  _full_spec((FP, FP)), _full_spec((1, FP)), _full_spec((1, FP))],
        out_specs=[_row_spec(), _vec_spec(), _vec_spec()],
        out_shape=[
            jax.ShapeDtypeStruct((NP, FP), jnp.float32),
            jax.ShapeDtypeStruct((NP,), jnp.float32),
            jax.ShapeDtypeStruct((NP,), jnp.float32),
        ],
    )(raw, s, b2, wp, av2, ad2)


QP = 608


def _pool1(raw, s, b2, wt, qt, batchp):
    return pl.pallas_call(
        _pool1_body,
        grid=(GRID_N,),
        in_specs=[_row_spec(), _vec_spec(), _full_spec((1, FP)),
                  _full_spec((FP, QP)), _full_spec((QP, B)), _vec_spec()],
        out_specs=[_row_spec(), _vec_spec(),
                   pl.BlockSpec((1, B), lambda i: (0, 0))],
        out_shape=[
            jax.ShapeDtypeStruct((NP, FP), jnp.float32),
            jax.ShapeDtypeStruct((NP,), jnp.float32),
            jax.ShapeDtypeStruct((1, B), jnp.float32),
        ],
        scratch_shapes=[pltpu.VMEM((FP, B), jnp.float32)],
    )(raw, s, b2, wt, qt, batchp)


CP = 128


def _pool2(h3, t, batchp, m, wl, bl2):
    return pl.pallas_call(
        _pool2_body,
        grid=(GRID_N,),
        in_specs=[_row_spec(), _vec_spec(), _vec_spec(),
                  _full_spec((1, B)), _full_spec((FP, CP)), _full_spec((1, CP))],
        out_specs=pl.BlockSpec((B, CP), lambda i: (0, 0)),
        out_shape=jax.ShapeDtypeStruct((B, CP), jnp.float32),
        scratch_shapes=[pltpu.VMEM((B, FP), jnp.float32),
                        pltpu.VMEM((B, 1), jnp.float32)],
    )(h3, t, batchp, m, wl, bl2)


def _to_half_space(v):
    return v + 120 * (v >= HALF).astype(v.dtype)


def _pad_rows(a):
    z = jnp.zeros((HALFP - HALF,) + a.shape[1:], a.dtype)
    return jnp.concatenate([a[:HALF], z, a[HALF:], z], axis=0)


def kernel(x, edges, query, batch, W_gat, att_src, att_dst, b_gat, W_att, W_lin, b_lin):
    f32 = jnp.float32
    xc = jnp.pad(x, ((0, 0), (0, FP - F_FEAT)))
    xp = _pad_rows(xc)

    ar = jnp.arange(N, dtype=edges.dtype)
    src = _to_half_space(jnp.concatenate([edges[0], ar]))
    dst = _to_half_space(jnp.concatenate([edges[1], ar]))
    pad_e = EPP - EP
    srcp = jnp.concatenate([src, jnp.zeros((pad_e,), edges.dtype)])
    dstp = jnp.concatenate([dst, jnp.full((pad_e,), NP, edges.dtype)])

    batchp = _pad_rows(batch[:, None]).squeeze(-1)
    batchp = jnp.where(
        (jnp.arange(NP) % HALFP) < HALF, batchp, jnp.int32(B))

    wp = jnp.pad(W_gat, ((0, FP - F_FEAT), (0, FP - F_FEAT))).T
    av2 = jnp.pad(att_src, (0, FP - F_FEAT))[None, :]
    ad2 = jnp.pad(att_dst, (0, FP - F_FEAT))[None, :]
    b2 = jnp.pad(b_gat, (0, FP - F_FEAT))[None, :]
    wt = jnp.pad(W_att, ((0, QP - Q_DIM), (0, FP - F_FEAT))).T
    qt = jnp.pad(query, ((0, 0), (0, QP - Q_DIM))).T
    wl = jnp.pad(W_lin, ((0, CP - CLASSES), (0, FP - F_FEAT))).T
    bl2 = jnp.pad(b_lin, (0, CP - CLASSES))[None, :]

    h, a_s, a_d = _mm_first(xp, wp, av2, ad2)
    epk = dstp * jnp.int32(16384) + srcp
    for layer in range(3):
        raw, s, _ce = _sc_edge(epk, a_s, a_d, h)
        if layer < 2:
            h, a_s, a_d = _mm_next(raw, s, b2, wp, av2, ad2)

    h3, t, m = _pool1(raw, s, b2, wt, qt, batchp)
    logits = _pool2(h3, t, batchp, m, wl, bl2)
    return logits[:, :CLASSES].astype(f32)


# speedup vs baseline: 11.7515x; 1.1866x over previous
import functools

import jax
import jax.numpy as jnp
import numpy as np
from jax import lax
from jax.experimental import pallas as pl
from jax.experimental.pallas import tpu as pltpu
from jax.experimental.pallas import tpu_sc as plsc

N = 10000
E = 160000
F_FEAT = 302
Q_DIM = 600
B = 16
CLASSES = 100

FP = 304
HALF = 5000
HALFP = 5120
NP = 2 * HALFP
EP = E + N
CHUNK = 10640
EPP = 16 * CHUNK
NB_RED = HALFP // (16 * 16)
SEG = HALFP // 16
SUB = 2128
RT = 512
GRID_N = NP // RT


def _shift_up(v, sh):
    j = jnp.maximum(lax.iota(jnp.int32, 16) - sh, 0)
    return v.at[j].get(mode="promise_in_bounds")


OUTR = 5008
SEG5 = OUTR // 16
ROWS_PER_T = 313


def _sc_edge_body(ep_hbm, as_hbm, ad_hbm, h_hbm,
                  out_hbm, ce_hbm,
                  es_pack, cnts,
                  as_loc, ad_loc, macc, tmpseg, red320,
                  rowbuf, rowbuf2, sidx, sidx2, didx,
                  out_sh, red_small,
                  sem, sem2):
    c = lax.axis_index("c")
    t = lax.axis_index("s")
    cbase = c * HALFP
    seg0 = t * SEG
    cebase = (c * 16 + t) * CHUNK
    iota16 = lax.iota(jnp.int32, 16)
    f32 = jnp.float32

    pltpu.sync_copy(as_hbm, as_loc)
    pltpu.sync_copy(ad_hbm.at[pl.ds(cbase, HALFP)], ad_loc)

    def _zneg(i, _):
        macc[pl.ds(i * 16, 16)] = jnp.full((16,), -1e30, f32)
        return 0
    lax.fori_loop(0, HALFP // 16, _zneg, 0)

    def _zrow(i, _):
        rowbuf[i // (FP // 16), pl.ds((i % (FP // 16)) * 16, 16)] = jnp.zeros((16,), f32)
        return 0
    lax.fori_loop(0, 16 * (FP // 16), _zrow, 0)

    @pl.when(t < 7)
    def _():
        pltpu.sync_copy(rowbuf, out_hbm.at[pl.ds(cbase + OUTR + t * 16, 16)])

    nz = jnp.minimum(20, SEG5 - t * 20)

    def _zout(k, _):
        pltpu.sync_copy(rowbuf, out_sh.at[pl.ds((t * 20 + k) * 16, 16)])
        return 0
    lax.fori_loop(0, nz, _zout, 0)

    def _psub(pno, _):
        pltpu.sync_copy(ep_hbm.at[pl.ds(t * CHUNK + pno * SUB, SUB)], es_pack)

        def _part(blk, fc):
            pk = es_pack[pl.ds(blk * 16, 16)]
            dl = (pk >> 14) - cbase
            msk = (dl >= 0) & (dl < HALFP)
            mi = msk.astype(jnp.int32)
            pos = fc + plsc.cumsum(mi) - 1
            plsc.store_scatter(es_pack, [pos], pk, mask=msk)
            return fc + jnp.sum(mi)
        fc = lax.fori_loop(0, SUB // 16, _part, jnp.int32(0))
        plsc.store_scatter(cnts, [jnp.full((16,), pno, jnp.int32)],
                           jnp.full((16,), fc, jnp.int32), mask=iota16 == 0)
        pltpu.sync_copy(es_pack, ce_hbm.at[pl.ds(cebase + pno * SUB, SUB)])
        return 0
    lax.fori_loop(0, CHUNK // SUB, _psub, 0)
    cvec = cnts[...]

    def _p2sub(pno, _):
        fcp = jnp.sum(jnp.where(iota16 == pno, cvec, 0))
        pltpu.sync_copy(ce_hbm.at[pl.ds(cebase + pno * SUB, SUB)], es_pack)

        def _p2(blk, _):
            off = blk * 16
            lm = (off + iota16) < fcp
            pk = es_pack[pl.ds(off, 16)]
            pk = jnp.where(lm, pk, cbase * 16384)
            sl = pk & 16383
            dl = (pk >> 14) - cbase
            av = plsc.load_gather(as_loc, [sl])
            ks, vs, om = plsc.sort_key_val(dl, av, mask=lm)
            for sh in (1, 2, 4, 8):
                kj = _shift_up(ks, sh)
                vj = _shift_up(vs, sh)
                same = (kj == ks) & (iota16 >= sh)
                vs = jnp.where(same, jnp.maximum(vs, vj), vs)
            _, lastm = plsc.scan_count(ks, mask=om)
            kw = jnp.where(lastm, ks, 0)
            cur = plsc.load_gather(macc, [kw])
            plsc.store_scatter(macc, [kw], jnp.maximum(cur, vs), mask=lastm)
            return 0
        lax.fori_loop(0, (fcp + 15) // 16, _p2, 0)
        return 0
    lax.fori_loop(0, CHUNK // SUB, _p2sub, 0)

    def _zr(i, _):
        red320[pl.ds(i * 16, 16)] = jnp.full((16,), -1e30, f32)
        return 0
    lax.fori_loop(0, NB_RED, _zr, 0)

    def _rring(r, _):
        sseg = lax.rem(t + r, 16)
        pltpu.sync_copy(macc.at[pl.ds(sseg * SEG, SEG)],
                        red_small.at[pl.ds(sseg * SEG, SEG)])
        plsc.subcore_barrier()
        pltpu.sync_copy(red_small.at[pl.ds(t * SEG, SEG)], tmpseg)

        def _acc(j, _):
            col = pl.ds(j * 16, 16)
            red320[col] = jnp.maximum(red320[col], tmpseg[col])
            return 0
        lax.fori_loop(0, NB_RED, _acc, 0)
        plsc.subcore_barrier()
        return 0
    lax.fori_loop(0, 16, _rring, 0)

    def _mfin(j, _):
        col = pl.ds(j * 16, 16)
        x = ad_loc[pl.ds(seg0 + j * 16, 16)] + red320[col]
        red320[col] = jnp.where(x >= 0, x, 0.2 * x)
        return 0
    lax.fori_loop(0, NB_RED, _mfin, 0)
    plsc.subcore_barrier()
    pltpu.sync_copy(red320, red_small.at[pl.ds(seg0, SEG)])
    plsc.subcore_barrier()
    pltpu.sync_copy(red_small, macc)

    cpad = cbase * 16384

    def _blkpk(off, fcp):
        lm = (off + iota16) < fcp
        pk = es_pack[pl.ds(off, 16)]
        return jnp.where(lm, pk, cpad), lm

    def _p4sub(pno, _):
        fcp = jnp.sum(jnp.where(iota16 == pno, cvec, 0))
        pltpu.sync_copy(ce_hbm.at[pl.ds(cebase + pno * SUB, SUB)], es_pack)
        npair = (((fcp + 15) // 16) + 1) // 2

        pk0, _lm0 = _blkpk(0, fcp)
        sidx[...] = pk0 & 16383
        pltpu.async_copy(h_hbm.at[sidx], rowbuf, sem)

        def _pair(g, _):
            for slot in range(2):
                b = g * 2 + slot
                rb, rb_n = (rowbuf, rowbuf2) if slot == 0 else (rowbuf2, rowbuf)
                si, si_n = (sidx, sidx2) if slot == 0 else (sidx2, sidx)
                se, se_n = (sem, sem2) if slot == 0 else (sem2, sem)
                off = b * 16
                pk, lm = _blkpk(off, fcp)
                sl = pk & 16383
                dl = (pk >> 14) - cbase
                a1 = plsc.load_gather(as_loc, [sl])
                a2 = plsc.load_gather(ad_loc, [dl])
                mm = plsc.load_gather(macc, [dl])
                xx = a1 + a2
                e = jnp.where(xx >= 0, xx, 0.2 * xx)
                ex = jnp.where(lm, jnp.exp(e - mm), 0.0)
                offn = jnp.minimum(off + 16, SUB - 16)
                pkn, _lmn = _blkpk(offn, fcp)
                si_n[...] = pkn & 16383
                pltpu.async_copy(h_hbm.at[si_n], rb_n, se_n)
                pltpu.make_async_copy(h_hbm.at[si], rb, se).wait()
                didx[...] = dl
                for r in range(16):
                    er = ex[r]
                    for jj in range(FP // 16):
                        slc = pl.ds(jj * 16, 16)
                        rb[r, slc] = rb[r, slc] * er
                pltpu.sync_copy(rb, out_sh.at[didx], add=True)
            return 0
        lax.fori_loop(0, npair, _pair, 0)
        pltpu.make_async_copy(h_hbm.at[sidx], rowbuf, sem).wait()
        return 0
    lax.fori_loop(0, CHUNK // SUB, _p4sub, 0)

    plsc.subcore_barrier()
    pltpu.sync_copy(out_sh.at[pl.ds(t * ROWS_PER_T, ROWS_PER_T)],
                    out_hbm.at[pl.ds(cbase + t * ROWS_PER_T, ROWS_PER_T)])


_sc_edge = functools.partial(
    pl.kernel,
    out_type=[
        jax.ShapeDtypeStruct((NP, FP), jnp.float32),
        jax.ShapeDtypeStruct((32 * CHUNK,), jnp.int32),
    ],
    mesh=plsc.VectorSubcoreMesh(
        core_axis_name="c", subcore_axis_name="s", num_cores=2, num_subcores=16
    ),
    compiler_params=pltpu.CompilerParams(
        needs_layout_passes=False, use_tc_tiling_on_sc=False),
    scratch_types=[
        pltpu.VMEM((SUB,), jnp.int32),
        pltpu.VMEM((16,), jnp.int32),
        pltpu.VMEM((NP,), jnp.float32),
        pltpu.VMEM((HALFP,), jnp.float32),
        pltpu.VMEM((HALFP,), jnp.float32),
        pltpu.VMEM((SEG,), jnp.float32),
        pltpu.VMEM((SEG,), jnp.float32),
        pltpu.VMEM((16, FP), jnp.float32),
        pltpu.VMEM((16, FP), jnp.float32),
        pltpu.VMEM((16,), jnp.int32),
        pltpu.VMEM((16,), jnp.int32),
        pltpu.VMEM((16,), jnp.int32),
        pltpu.VMEM_SHARED((OUTR, FP), jnp.float32),
        pltpu.VMEM_SHARED((16 * SEG,), jnp.float32),
        pltpu.SemaphoreType.DMA,
        pltpu.SemaphoreType.DMA,
    ],
)(_sc_edge_body)


def _mm_first_body(x_ref, w_ref, av_ref, ad_ref, h_ref, as_ref, ad_out_ref):
    y = jnp.dot(x_ref[...], w_ref[...], preferred_element_type=jnp.float32)
    col = lax.broadcasted_iota(jnp.int32, (RT, FP), 1)
    h_ref[...] = jnp.where(col == F_FEAT, 1.0, y)
    as_ref[...] = jnp.sum(y * av_ref[...], axis=1)
    ad_out_ref[...] = jnp.sum(y * ad_ref[...], axis=1)


def _mm_next_body(raw_ref, b_ref, w_ref, av_ref, ad_ref,
                  h_ref, as_ref, ad_out_ref):
    raw = raw_ref[...]
    col = lax.broadcasted_iota(jnp.int32, (RT, FP), 1)
    sv = jnp.sum(jnp.where(col == F_FEAT, raw, 0.0), axis=1)
    hin = jnp.maximum(raw / (sv[:, None] + 1e-16) + b_ref[...], 0.0)
    y = jnp.dot(hin, w_ref[...], preferred_element_type=jnp.float32)
    h_ref[...] = jnp.where(col == F_FEAT, 1.0, y)
    as_ref[...] = jnp.sum(y * av_ref[...], axis=1)
    ad_out_ref[...] = jnp.sum(y * ad_ref[...], axis=1)


def _pool1_body(raw_ref, b_ref, wt_ref, qt_ref, batch_ref,
                h3_ref, t_ref, m_ref, qw_scratch):
    i = pl.program_id(0)

    @pl.when(i == 0)
    def _():
        qw_scratch[...] = jnp.dot(
            wt_ref[...], qt_ref[...], preferred_element_type=jnp.float32
        ) * (1.0 / np.sqrt(Q_DIM))

    raw = raw_ref[...]
    colp = lax.broadcasted_iota(jnp.int32, (RT, FP), 1)
    sv = jnp.sum(jnp.where(colp == F_FEAT, raw, 0.0), axis=1)
    hin = jnp.maximum(raw / (sv[:, None] + 1e-16) + b_ref[...], 0.0)
    h3_ref[...] = hin
    attq = jnp.dot(hin, qw_scratch[...], preferred_element_type=jnp.float32)
    gids = lax.broadcasted_iota(jnp.int32, (RT, B), 1)
    oh = batch_ref[...][:, None] == gids
    t_ref[...] = jnp.sum(jnp.where(oh, attq, 0.0), axis=1)
    mt = jnp.max(jnp.where(oh, attq, -3.0e38), axis=0, keepdims=True)

    @pl.when(i == 0)
    def _():
        m_ref[...] = jnp.full((1, B), -3.0e38, jnp.float32)

    m_ref[...] = jnp.maximum(m_ref[...], mt)


def _pool2_body(h3_ref, t_ref, batch_ref, m_ref, wl_ref, bl_ref,
                logits_ref, pooled_sc, ssum_sc):
    i = pl.program_id(0)

    @pl.when(i == 0)
    def _():
        pooled_sc[...] = jnp.zeros((B, FP), jnp.float32)
        ssum_sc[...] = jnp.zeros((B, 1), jnp.float32)

    gids = lax.broadcasted_iota(jnp.int32, (B, RT), 0)
    ohf = (gids == batch_ref[...][None, :]).astype(jnp.float32)
    m_n = jnp.sum(ohf * m_ref[...].reshape(B, 1), axis=0)
    w_u = jnp.exp(t_ref[...] - m_n)
    ohw = ohf * w_u[None, :]
    ssum_sc[...] += jnp.sum(ohw, axis=1, keepdims=True)
    pooled_sc[...] += jnp.dot(ohw, h3_ref[...], preferred_element_type=jnp.float32)

    @pl.when(i == pl.num_programs(0) - 1)
    def _():
        pooled = jnp.maximum(pooled_sc[...] / (ssum_sc[...] + 1e-16), 0.0)
        logits_ref[...] = jnp.dot(
            pooled, wl_ref[...], preferred_element_type=jnp.float32
        ) + bl_ref[...]


def _row_spec():
    return pl.BlockSpec((RT, FP), lambda i: (i, 0))


def _vec_spec():
    return pl.BlockSpec((RT,), lambda i: (i,))


def _full_spec(shape):
    nd = len(shape)
    return pl.BlockSpec(shape, lambda i: (0,) * nd)


def _mm_first(xp, wp, av2, ad2):
    return pl.pallas_call(
        _mm_first_body,
        grid=(GRID_N,),
        in_specs=[_row_spec(), _full_spec((FP, FP)), _full_spec((1, FP)),
                  _full_spec((1, FP))],
        out_specs=[_row_spec(), _vec_spec(), _vec_spec()],
        out_shape=[
            jax.ShapeDtypeStruct((NP, FP), jnp.float32),
            jax.ShapeDtypeStruct((NP,), jnp.float32),
            jax.ShapeDtypeStruct((NP,), jnp.float32),
        ],
    )(xp, wp, av2, ad2)


def _mm_next(raw, b2, wp, av2, ad2):
    return pl.pallas_call(
        _mm_next_body,
        grid=(GRID_N,),
        in_specs=[_row_spec(), _full_spec((1, FP)),
                  _full_spec((FP, FP)), _full_spec((1, FP)), _full_spec((1, FP))],
        out_specs=[_row_spec(), _vec_spec(), _vec_spec()],
        out_shape=[
            jax.ShapeDtypeStruct((NP, FP), jnp.float32),
            jax.ShapeDtypeStruct((NP,), jnp.float32),
            jax.ShapeDtypeStruct((NP,), jnp.float32),
        ],
    )(raw, b2, wp, av2, ad2)


QP = 608


def _pool1(raw, b2, wt, qt, batchp):
    return pl.pallas_call(
        _pool1_body,
        grid=(GRID_N,),
        in_specs=[_row_spec(), _full_spec((1, FP)),
                  _full_spec((FP, QP)), _full_spec((QP, B)), _vec_spec()],
        out_specs=[_row_spec(), _vec_spec(),
                   pl.BlockSpec((1, B), lambda i: (0, 0))],
        out_shape=[
            jax.ShapeDtypeStruct((NP, FP), jnp.float32),
            jax.ShapeDtypeStruct((NP,), jnp.float32),
            jax.ShapeDtypeStruct((1, B), jnp.float32),
        ],
        scratch_shapes=[pltpu.VMEM((FP, B), jnp.float32)],
    )(raw, b2, wt, qt, batchp)


CP = 128


def _pool2(h3, t, batchp, m, wl, bl2):
    return pl.pallas_call(
        _pool2_body,
        grid=(GRID_N,),
        in_specs=[_row_spec(), _vec_spec(), _vec_spec(),
                  _full_spec((1, B)), _full_spec((FP, CP)), _full_spec((1, CP))],
        out_specs=pl.BlockSpec((B, CP), lambda i: (0, 0)),
        out_shape=jax.ShapeDtypeStruct((B, CP), jnp.float32),
        scratch_shapes=[pltpu.VMEM((B, FP), jnp.float32),
                        pltpu.VMEM((B, 1), jnp.float32)],
    )(h3, t, batchp, m, wl, bl2)


def _to_half_space(v):
    return v + 120 * (v >= HALF).astype(v.dtype)


def _pad_rows(a):
    z = jnp.zeros((HALFP - HALF,) + a.shape[1:], a.dtype)
    return jnp.concatenate([a[:HALF], z, a[HALF:], z], axis=0)


def kernel(x, edges, query, batch, W_gat, att_src, att_dst, b_gat, W_att, W_lin, b_lin):
    f32 = jnp.float32
    xc = jnp.pad(x, ((0, 0), (0, FP - F_FEAT)))
    xp = _pad_rows(xc)

    ar = jnp.arange(N, dtype=edges.dtype)
    src = _to_half_space(jnp.concatenate([edges[0], ar]))
    dst = _to_half_space(jnp.concatenate([edges[1], ar]))
    pad_e = EPP - EP
    srcp = jnp.concatenate([src, jnp.zeros((pad_e,), edges.dtype)])
    dstp = jnp.concatenate([dst, jnp.full((pad_e,), NP, edges.dtype)])

    batchp = _pad_rows(batch[:, None]).squeeze(-1)
    batchp = jnp.where(
        (jnp.arange(NP) % HALFP) < HALF, batchp, jnp.int32(B))

    wp = jnp.pad(W_gat, ((0, FP - F_FEAT), (0, FP - F_FEAT))).T
    av2 = jnp.pad(att_src, (0, FP - F_FEAT))[None, :]
    ad2 = jnp.pad(att_dst, (0, FP - F_FEAT))[None, :]
    b2 = jnp.pad(b_gat, (0, FP - F_FEAT))[None, :]
    wt = jnp.pad(W_att, ((0, QP - Q_DIM), (0, FP - F_FEAT))).T
    qt = jnp.pad(query, ((0, 0), (0, QP - Q_DIM))).T
    wl = jnp.pad(W_lin, ((0, CP - CLASSES), (0, FP - F_FEAT))).T
    bl2 = jnp.pad(b_lin, (0, CP - CLASSES))[None, :]

    h, a_s, a_d = _mm_first(xp, wp, av2, ad2)
    epk = dstp * jnp.int32(16384) + srcp
    for layer in range(3):
        raw, _ce = _sc_edge(epk, a_s, a_d, h)
        if layer < 2:
            h, a_s, a_d = _mm_next(raw, b2, wp, av2, ad2)

    h3, t, m = _pool1(raw, b2, wt, qt, batchp)
    logits = _pool2(h3, t, batchp, m, wl, bl2)
    return logits[:, :CLASSES].astype(f32)


# speedup vs baseline: 11.9689x; 1.0185x over previous
import functools

import jax
import jax.numpy as jnp
import numpy as np
from jax import lax
from jax.experimental import pallas as pl
from jax.experimental.pallas import tpu as pltpu
from jax.experimental.pallas import tpu_sc as plsc

N = 10000
E = 160000
F_FEAT = 302
Q_DIM = 600
B = 16
CLASSES = 100

FP = 304
HALF = 5000
HALFP = 5120
NP = 2 * HALFP
EP = E + N
CHUNK = 10640
EPP = 16 * CHUNK
NB_RED = HALFP // (16 * 16)
SEG = HALFP // 16
SUB = 2128
RT = 512
GRID_N = NP // RT


def _shift_up(v, sh):
    j = jnp.maximum(lax.iota(jnp.int32, 16) - sh, 0)
    return v.at[j].get(mode="promise_in_bounds")


OUTR = 5008
SEG5 = OUTR // 16
ROWS_PER_T = 313


def _sc_edge_body(do_part,
                  ep_hbm, as_hbm, ad_hbm, h_hbm,
                  out_hbm, ce_hbm,
                  es_pack, cnts,
                  as_loc, ad_loc, macc, tmpseg, red320,
                  rowbuf, rowbuf2, sidx, sidx2, didx,
                  out_sh, red_small,
                  sem, sem2, ssem, ssem2):
    c = lax.axis_index("c")
    t = lax.axis_index("s")
    cbase = c * HALFP
    seg0 = t * SEG
    cebase = (c * 16 + t) * (CHUNK + 16)
    iota16 = lax.iota(jnp.int32, 16)
    f32 = jnp.float32

    pltpu.sync_copy(as_hbm, as_loc)
    pltpu.sync_copy(ad_hbm.at[pl.ds(cbase, HALFP)], ad_loc)

    def _zneg(i, _):
        macc[pl.ds(i * 16, 16)] = jnp.full((16,), -1e30, f32)
        return 0
    lax.fori_loop(0, HALFP // 16, _zneg, 0)

    def _zrow(i, _):
        rowbuf[i // (FP // 16), pl.ds((i % (FP // 16)) * 16, 16)] = jnp.zeros((16,), f32)
        return 0
    lax.fori_loop(0, 16 * (FP // 16), _zrow, 0)

    @pl.when(t < 7)
    def _():
        pltpu.sync_copy(rowbuf, out_hbm.at[pl.ds(cbase + OUTR + t * 16, 16)])

    nz = jnp.minimum(20, SEG5 - t * 20)

    def _zout(k, _):
        pltpu.sync_copy(rowbuf, out_sh.at[pl.ds((t * 20 + k) * 16, 16)])
        return 0
    lax.fori_loop(0, nz, _zout, 0)

    if do_part:
        def _psub(pno, _):
            pltpu.sync_copy(ep_hbm.at[pl.ds(t * CHUNK + pno * SUB, SUB)], es_pack)

            def _part(blk, fc):
                pk = es_pack[pl.ds(blk * 16, 16)]
                dl = (pk >> 14) - cbase
                msk = (dl >= 0) & (dl < HALFP)
                mi = msk.astype(jnp.int32)
                pos = fc + plsc.cumsum(mi) - 1
                plsc.store_scatter(es_pack, [pos], pk, mask=msk)
                return fc + jnp.sum(mi)
            fc = lax.fori_loop(0, SUB // 16, _part, jnp.int32(0))
            plsc.store_scatter(cnts, [jnp.full((16,), pno, jnp.int32)],
                               jnp.full((16,), fc, jnp.int32), mask=iota16 == 0)
            pltpu.sync_copy(es_pack, ce_hbm.at[pl.ds(cebase + pno * SUB, SUB)])
            return 0
        lax.fori_loop(0, CHUNK // SUB, _psub, 0)
        pltpu.sync_copy(cnts, ce_hbm.at[pl.ds(cebase + CHUNK, 16)])
    else:
        pltpu.sync_copy(ep_hbm.at[pl.ds(cebase + CHUNK, 16)], cnts)
    cvec = cnts[...]
    ce_src_hbm = ce_hbm if do_part else ep_hbm

    def _p2sub(pno, _):
        fcp = jnp.sum(jnp.where(iota16 == pno, cvec, 0))
        pltpu.sync_copy(ce_src_hbm.at[pl.ds(cebase + pno * SUB, SUB)], es_pack)

        def _p2(blk, _):
            off = blk * 16
            lm = (off + iota16) < fcp
            pk = es_pack[pl.ds(off, 16)]
            pk = jnp.where(lm, pk, cbase * 16384)
            sl = pk & 16383
            dl = (pk >> 14) - cbase
            av = plsc.load_gather(as_loc, [sl])
            ks, vs, om = plsc.sort_key_val(dl, av, mask=lm)
            for sh in (1, 2, 4, 8):
                kj = _shift_up(ks, sh)
                vj = _shift_up(vs, sh)
                same = (kj == ks) & (iota16 >= sh)
                vs = jnp.where(same, jnp.maximum(vs, vj), vs)
            _, lastm = plsc.scan_count(ks, mask=om)
            kw = jnp.where(lastm, ks, 0)
            cur = plsc.load_gather(macc, [kw])
            plsc.store_scatter(macc, [kw], jnp.maximum(cur, vs), mask=lastm)
            return 0
        lax.fori_loop(0, (fcp + 15) // 16, _p2, 0)
        return 0
    lax.fori_loop(0, CHUNK // SUB, _p2sub, 0)

    def _zr(i, _):
        red320[pl.ds(i * 16, 16)] = jnp.full((16,), -1e30, f32)
        return 0
    lax.fori_loop(0, NB_RED, _zr, 0)

    def _rring(r, _):
        sseg = lax.rem(t + r, 16)
        pltpu.sync_copy(macc.at[pl.ds(sseg * SEG, SEG)],
                        red_small.at[pl.ds(sseg * SEG, SEG)])
        plsc.subcore_barrier()
        pltpu.sync_copy(red_small.at[pl.ds(t * SEG, SEG)], tmpseg)

        def _acc(j, _):
            col = pl.ds(j * 16, 16)
            red320[col] = jnp.maximum(red320[col], tmpseg[col])
            return 0
        lax.fori_loop(0, NB_RED, _acc, 0)
        plsc.subcore_barrier()
        return 0
    lax.fori_loop(0, 16, _rring, 0)

    def _mfin(j, _):
        col = pl.ds(j * 16, 16)
        x = ad_loc[pl.ds(seg0 + j * 16, 16)] + red320[col]
        red320[col] = jnp.where(x >= 0, x, 0.2 * x)
        return 0
    lax.fori_loop(0, NB_RED, _mfin, 0)
    plsc.subcore_barrier()
    pltpu.sync_copy(red320, red_small.at[pl.ds(seg0, SEG)])
    plsc.subcore_barrier()
    pltpu.sync_copy(red_small, macc)

    cpad = cbase * 16384

    def _blkpk(off, fcp):
        lm = (off + iota16) < fcp
        pk = es_pack[pl.ds(off, 16)]
        return jnp.where(lm, pk, cpad), lm

    def _p4sub(pno, _):
        fcp = jnp.sum(jnp.where(iota16 == pno, cvec, 0))
        pltpu.sync_copy(ce_src_hbm.at[pl.ds(cebase + pno * SUB, SUB)], es_pack)
        npair = jnp.maximum((((fcp + 15) // 16) + 1) // 2, 1)

        pk0, _lm0 = _blkpk(0, fcp)
        sidx[...] = pk0 & 16383
        pltpu.async_copy(h_hbm.at[sidx], rowbuf, sem)

        def _pair(g, _):
            for slot in range(2):
                b = g * 2 + slot
                rb, rb_n = (rowbuf, rowbuf2) if slot == 0 else (rowbuf2, rowbuf)
                si, si_n = (sidx, sidx2) if slot == 0 else (sidx2, sidx)
                se, se_n = (sem, sem2) if slot == 0 else (sem2, sem)
                ss, ss_n = (ssem, ssem2) if slot == 0 else (ssem2, ssem)
                off = b * 16
                pk, lm = _blkpk(off, fcp)
                sl = pk & 16383
                dl = (pk >> 14) - cbase
                a1 = plsc.load_gather(as_loc, [sl])
                a2 = plsc.load_gather(ad_loc, [dl])
                mm = plsc.load_gather(macc, [dl])
                xx = a1 + a2
                e = jnp.where(xx >= 0, xx, 0.2 * xx)
                ex = jnp.where(lm, jnp.exp(e - mm), 0.0)
                offn = jnp.minimum(off + 16, SUB - 16)
                pkn, _lmn = _blkpk(offn, fcp)
                if slot == 1:
                    pltpu.make_async_copy(rb_n, out_sh.at[didx], ss_n).wait()
                else:
                    @pl.when(g > 0)
                    def _():
                        pltpu.make_async_copy(rb_n, out_sh.at[didx], ss_n).wait()
                si_n[...] = pkn & 16383
                pltpu.async_copy(h_hbm.at[si_n], rb_n, se_n)
                pltpu.make_async_copy(h_hbm.at[si], rb, se).wait()
                didx[...] = dl
                for r in range(16):
                    er = ex[r]
                    for jj in range(FP // 16):
                        slc = pl.ds(jj * 16, 16)
                        rb[r, slc] = rb[r, slc] * er
                pltpu.async_copy(rb, out_sh.at[didx], ss, add=True)
            return 0
        lax.fori_loop(0, npair, _pair, 0)
        pltpu.make_async_copy(h_hbm.at[sidx], rowbuf, sem).wait()
        pltpu.make_async_copy(rowbuf2, out_sh.at[didx], ssem2).wait()
        return 0
    lax.fori_loop(0, CHUNK // SUB, _p4sub, 0)

    plsc.subcore_barrier()
    pltpu.sync_copy(out_sh.at[pl.ds(t * ROWS_PER_T, ROWS_PER_T)],
                    out_hbm.at[pl.ds(cbase + t * ROWS_PER_T, ROWS_PER_T)])


_sc_edge = functools.partial(
    pl.kernel,
    out_type=[
        jax.ShapeDtypeStruct((NP, FP), jnp.float32),
        jax.ShapeDtypeStruct((32 * (CHUNK + 16),), jnp.int32),
    ],
    mesh=plsc.VectorSubcoreMesh(
        core_axis_name="c", subcore_axis_name="s", num_cores=2, num_subcores=16
    ),
    compiler_params=pltpu.CompilerParams(
        needs_layout_passes=False, use_tc_tiling_on_sc=False),
    scratch_types=[
        pltpu.VMEM((SUB,), jnp.int32),
        pltpu.VMEM((16,), jnp.int32),
        pltpu.VMEM((NP,), jnp.float32),
        pltpu.VMEM((HALFP,), jnp.float32),
        pltpu.VMEM((HALFP,), jnp.float32),
        pltpu.VMEM((SEG,), jnp.float32),
        pltpu.VMEM((SEG,), jnp.float32),
        pltpu.VMEM((16, FP), jnp.float32),
        pltpu.VMEM((16, FP), jnp.float32),
        pltpu.VMEM((16,), jnp.int32),
        pltpu.VMEM((16,), jnp.int32),
        pltpu.VMEM((16,), jnp.int32),
        pltpu.VMEM_SHARED((OUTR, FP), jnp.float32),
        pltpu.VMEM_SHARED((16 * SEG,), jnp.float32),
        pltpu.SemaphoreType.DMA,
        pltpu.SemaphoreType.DMA,
        pltpu.SemaphoreType.DMA,
        pltpu.SemaphoreType.DMA,
    ],
)
_sc_edge_first = _sc_edge(functools.partial(_sc_edge_body, True))
_sc_edge_next = _sc_edge(functools.partial(_sc_edge_body, False))


def _mm_first_body(x_ref, w_ref, av_ref, ad_ref, h_ref, as_ref, ad_out_ref):
    y = jnp.dot(x_ref[...], w_ref[...], preferred_element_type=jnp.float32)
    col = lax.broadcasted_iota(jnp.int32, (RT, FP), 1)
    h_ref[...] = jnp.where(col == F_FEAT, 1.0, y)
    as_ref[...] = jnp.sum(y * av_ref[...], axis=1)
    ad_out_ref[...] = jnp.sum(y * ad_ref[...], axis=1)


def _mm_next_body(raw_ref, b_ref, w_ref, av_ref, ad_ref,
                  h_ref, as_ref, ad_out_ref):
    raw = raw_ref[...]
    col = lax.broadcasted_iota(jnp.int32, (RT, FP), 1)
    sv = jnp.sum(jnp.where(col == F_FEAT, raw, 0.0), axis=1)
    hin = jnp.maximum(raw / (sv[:, None] + 1e-16) + b_ref[...], 0.0)
    y = jnp.dot(hin, w_ref[...], preferred_element_type=jnp.float32)
    h_ref[...] = jnp.where(col == F_FEAT, 1.0, y)
    as_ref[...] = jnp.sum(y * av_ref[...], axis=1)
    ad_out_ref[...] = jnp.sum(y * ad_ref[...], axis=1)


def _pool1_body(raw_ref, b_ref, wt_ref, qt_ref, batch_ref,
                h3_ref, t_ref, m_ref, qw_scratch):
    i = pl.program_id(0)

    @pl.when(i == 0)
    def _():
        qw_scratch[...] = jnp.dot(
            wt_ref[...], qt_ref[...], preferred_element_type=jnp.float32
        ) * (1.0 / np.sqrt(Q_DIM))

    raw = raw_ref[...]
    colp = lax.broadcasted_iota(jnp.int32, (RT, FP), 1)
    sv = jnp.sum(jnp.where(colp == F_FEAT, raw, 0.0), axis=1)
    hin = jnp.maximum(raw / (sv[:, None] + 1e-16) + b_ref[...], 0.0)
    h3_ref[...] = hin
    attq = jnp.dot(hin, qw_scratch[...], preferred_element_type=jnp.float32)
    gids = lax.broadcasted_iota(jnp.int32, (RT, B), 1)
    oh = batch_ref[...][:, None] == gids
    t_ref[...] = jnp.sum(jnp.where(oh, attq, 0.0), axis=1)
    mt = jnp.max(jnp.where(oh, attq, -3.0e38), axis=0, keepdims=True)

    @pl.when(i == 0)
    def _():
        m_ref[...] = jnp.full((1, B), -3.0e38, jnp.float32)

    m_ref[...] = jnp.maximum(m_ref[...], mt)


def _pool2_body(h3_ref, t_ref, batch_ref, m_ref, wl_ref, bl_ref,
                logits_ref, pooled_sc, ssum_sc):
    i = pl.program_id(0)

    @pl.when(i == 0)
    def _():
        pooled_sc[...] = jnp.zeros((B, FP), jnp.float32)
        ssum_sc[...] = jnp.zeros((B, 1), jnp.float32)

    gids = lax.broadcasted_iota(jnp.int32, (B, RT), 0)
    ohf = (gids == batch_ref[...][None, :]).astype(jnp.float32)
    m_n = jnp.sum(ohf * m_ref[...].reshape(B, 1), axis=0)
    w_u = jnp.exp(t_ref[...] - m_n)
    ohw = ohf * w_u[None, :]
    ssum_sc[...] += jnp.sum(ohw, axis=1, keepdims=True)
    pooled_sc[...] += jnp.dot(ohw, h3_ref[...], preferred_element_type=jnp.float32)

    @pl.when(i == pl.num_programs(0) - 1)
    def _():
        pooled = jnp.maximum(pooled_sc[...] / (ssum_sc[...] + 1e-16), 0.0)
        logits_ref[...] = jnp.dot(
            pooled, wl_ref[...], preferred_element_type=jnp.float32
        ) + bl_ref[...]


def _row_spec():
    return pl.BlockSpec((RT, FP), lambda i: (i, 0))


def _vec_spec():
    return pl.BlockSpec((RT,), lambda i: (i,))


def _full_spec(shape):
    nd = len(shape)
    return pl.BlockSpec(shape, lambda i: (0,) * nd)


def _mm_first(xp, wp, av2, ad2):
    return pl.pallas_call(
        _mm_first_body,
        grid=(GRID_N,),
        in_specs=[_row_spec(), _full_spec((FP, FP)), _full_spec((1, FP)),
                  _full_spec((1, FP))],
        out_specs=[_row_spec(), _vec_spec(), _vec_spec()],
        out_shape=[
            jax.ShapeDtypeStruct((NP, FP), jnp.float32),
            jax.ShapeDtypeStruct((NP,), jnp.float32),
            jax.ShapeDtypeStruct((NP,), jnp.float32),
        ],
    )(xp, wp, av2, ad2)


def _mm_next(raw, b2, wp, av2, ad2):
    return pl.pallas_call(
        _mm_next_body,
        grid=(GRID_N,),
        in_specs=[_row_spec(), _full_spec((1, FP)),
                  _full_spec((FP, FP)), _full_spec((1, FP)), _full_spec((1, FP))],
        out_specs=[_row_spec(), _vec_spec(), _vec_spec()],
        out_shape=[
            jax.ShapeDtypeStruct((NP, FP), jnp.float32),
            jax.ShapeDtypeStruct((NP,), jnp.float32),
            jax.ShapeDtypeStruct((NP,), jnp.float32),
        ],
    )(raw, b2, wp, av2, ad2)


QP = 608


def _pool1(raw, b2, wt, qt, batchp):
    return pl.pallas_call(
        _pool1_body,
        grid=(GRID_N,),
        in_specs=[_row_spec(), _full_spec((1, FP)),
                  _full_spec((FP, QP)), _full_spec((QP, B)), _vec_spec()],
        out_specs=[_row_spec(), _vec_spec(),
                   pl.BlockSpec((1, B), lambda i: (0, 0))],
        out_shape=[
            jax.ShapeDtypeStruct((NP, FP), jnp.float32),
            jax.ShapeDtypeStruct((NP,), jnp.float32),
            jax.ShapeDtypeStruct((1, B), jnp.float32),
        ],
        scratch_shapes=[pltpu.VMEM((FP, B), jnp.float32)],
    )(raw, b2, wt, qt, batchp)


CP = 128


def _pool2(h3, t, batchp, m, wl, bl2):
    return pl.pallas_call(
        _pool2_body,
        grid=(GRID_N,),
        in_specs=[_row_spec(), _vec_spec(), _vec_spec(),
                  _full_spec((1, B)), _full_spec((FP, CP)), _full_spec((1, CP))],
        out_specs=pl.BlockSpec((B, CP), lambda i: (0, 0)),
        out_shape=jax.ShapeDtypeStruct((B, CP), jnp.float32),
        scratch_shapes=[pltpu.VMEM((B, FP), jnp.float32),
                        pltpu.VMEM((B, 1), jnp.float32)],
    )(h3, t, batchp, m, wl, bl2)


def _to_half_space(v):
    return v + 120 * (v >= HALF).astype(v.dtype)


def _pad_rows(a):
    z = jnp.zeros((HALFP - HALF,) + a.shape[1:], a.dtype)
    return jnp.concatenate([a[:HALF], z, a[HALF:], z], axis=0)


def kernel(x, edges, query, batch, W_gat, att_src, att_dst, b_gat, W_att, W_lin, b_lin):
    f32 = jnp.float32
    xc = jnp.pad(x, ((0, 0), (0, FP - F_FEAT)))
    xp = _pad_rows(xc)

    ar = jnp.arange(N, dtype=edges.dtype)
    src = _to_half_space(jnp.concatenate([edges[0], ar]))
    dst = _to_half_space(jnp.concatenate([edges[1], ar]))
    pad_e = EPP - EP
    srcp = jnp.concatenate([src, jnp.zeros((pad_e,), edges.dtype)])
    dstp = jnp.concatenate([dst, jnp.full((pad_e,), NP, edges.dtype)])

    batchp = _pad_rows(batch[:, None]).squeeze(-1)
    batchp = jnp.where(
        (jnp.arange(NP) % HALFP) < HALF, batchp, jnp.int32(B))

    wp = jnp.pad(W_gat, ((0, FP - F_FEAT), (0, FP - F_FEAT))).T
    av2 = jnp.pad(att_src, (0, FP - F_FEAT))[None, :]
    ad2 = jnp.pad(att_dst, (0, FP - F_FEAT))[None, :]
    b2 = jnp.pad(b_gat, (0, FP - F_FEAT))[None, :]
    wt = jnp.pad(W_att, ((0, QP - Q_DIM), (0, FP - F_FEAT))).T
    qt = jnp.pad(query, ((0, 0), (0, QP - Q_DIM))).T
    wl = jnp.pad(W_lin, ((0, CP - CLASSES), (0, FP - F_FEAT))).T
    bl2 = jnp.pad(b_lin, (0, CP - CLASSES))[None, :]

    h, a_s, a_d = _mm_first(xp, wp, av2, ad2)
    epk = dstp * jnp.int32(16384) + srcp
    ce = None
    for layer in range(3):
        if layer == 0:
            raw, ce = _sc_edge_first(epk, a_s, a_d, h)
        else:
            raw, _ce2 = _sc_edge_next(ce, a_s, a_d, h)
        if layer < 2:
            h, a_s, a_d = _mm_next(raw, b2, wp, av2, ad2)

    h3, t, m = _pool1(raw, b2, wt, qt, batchp)
    logits = _pool2(h3, t, batchp, m, wl, bl2)
    return logits[:, :CLASSES].astype(f32)
